# Initial kernel scaffold; baseline (speedup 1.0000x reference)
import jax
import jax.numpy as jnp
from jax.experimental import pallas as pl


def kernel(x0, x1, x2, W_self0, W_neigh0, W_self1, W_neigh1):
    raise NotImplementedError("write your pallas kernel here")


# speedup vs baseline: 4.6227x; 4.6227x over previous
import jax
import jax.numpy as jnp
from jax.experimental import pallas as pl
from jax.experimental.pallas import tpu as pltpu

N = 10000
D = 128
K1 = 5
K2 = 10

B = 200
GRID = N // B


def _fused_body(x0_ref, x1_ref, x2_ref, ws0_ref, wn0_ref, ws1_ref, wn1_ref,
                out_ref):
    f32 = jnp.float32
    ws0 = ws0_ref[...]
    wn0 = wn0_ref[...]

    x1 = x1_ref[...]
    x2 = x2_ref[...]

    m2 = jnp.mean(x2.reshape(K1 * B, K2, D), axis=1)
    h1 = jnp.maximum(
        jnp.dot(x1, ws0, preferred_element_type=f32)
        + jnp.dot(m2, wn0, preferred_element_type=f32), 0.0)

    m1 = jnp.mean(x1.reshape(B, K1, D), axis=1)
    h0 = jnp.maximum(
        jnp.dot(x0_ref[...], ws0, preferred_element_type=f32)
        + jnp.dot(m1, wn0, preferred_element_type=f32), 0.0)

    mh1 = jnp.mean(h1.reshape(B, K1, D), axis=1)
    out_ref[...] = (
        jnp.dot(h0, ws1_ref[...], preferred_element_type=f32)
        + jnp.dot(mh1, wn1_ref[...], preferred_element_type=f32))


def kernel(x0, x1, x2, W_self0, W_neigh0, W_self1, W_neigh1):
    w_spec = pl.BlockSpec((D, D), lambda i: (0, 0))
    return pl.pallas_call(
        _fused_body,
        grid=(GRID,),
        in_specs=[
            pl.BlockSpec((B, D), lambda i: (i, 0)),
            pl.BlockSpec((K1 * B, D), lambda i: (i, 0)),
            pl.BlockSpec((K1 * K2 * B, D), lambda i: (i, 0)),
            w_spec, w_spec, w_spec, w_spec,
        ],
        out_specs=pl.BlockSpec((B, D), lambda i: (i, 0)),
        out_shape=jax.ShapeDtypeStruct((N, D), jnp.float32),
        compiler_params=pltpu.CompilerParams(
            dimension_semantics=("arbitrary",)),
    )(x0, x1, x2, W_self0, W_neigh0, W_self1, W_neigh1)


# speedup vs baseline: 11.3448x; 2.4541x over previous
import jax
import jax.numpy as jnp
from jax.experimental import pallas as pl
from jax.experimental.pallas import tpu as pltpu

N = 10000
D = 128
K1 = 5
K2 = 10

B = 200
GRID = N // B


def _fused_body(x0_ref, x1_ref, x2_ref, ws0_ref, wn0_ref, ws1_ref, wn1_ref,
                out_ref):
    f32 = jnp.float32
    ws0 = ws0_ref[...]
    wn0 = wn0_ref[...]

    m1 = None
    mh1 = None
    for j in range(K1):
        x1j = x1_ref[pl.Slice(j, B, K1), :]
        m2j = x2_ref[pl.Slice(K2 * j, B, K1 * K2), :]
        for u in range(1, K2):
            m2j = m2j + x2_ref[pl.Slice(K2 * j + u, B, K1 * K2), :]
        h1j = jnp.maximum(
            jnp.dot(x1j, ws0, preferred_element_type=f32)
            + jnp.dot(m2j * (1.0 / K2), wn0, preferred_element_type=f32),
            0.0)
        m1 = x1j if m1 is None else m1 + x1j
        mh1 = h1j if mh1 is None else mh1 + h1j

    h0 = jnp.maximum(
        jnp.dot(x0_ref[...], ws0, preferred_element_type=f32)
        + jnp.dot(m1 * (1.0 / K1), wn0, preferred_element_type=f32), 0.0)

    out_ref[...] = (
        jnp.dot(h0, ws1_ref[...], preferred_element_type=f32)
        + jnp.dot(mh1 * (1.0 / K1), wn1_ref[...], preferred_element_type=f32))


def kernel(x0, x1, x2, W_self0, W_neigh0, W_self1, W_neigh1):
    w_spec = pl.BlockSpec((D, D), lambda i: (0, 0))
    return pl.pallas_call(
        _fused_body,
        grid=(GRID,),
        in_specs=[
            pl.BlockSpec((B, D), lambda i: (i, 0)),
            pl.BlockSpec((K1 * B, D), lambda i: (i, 0)),
            pl.BlockSpec((K1 * K2 * B, D), lambda i: (i, 0)),
            w_spec, w_spec, w_spec, w_spec,
        ],
        out_specs=pl.BlockSpec((B, D), lambda i: (i, 0)),
        out_shape=jax.ShapeDtypeStruct((N, D), jnp.float32),
        compiler_params=pltpu.CompilerParams(
            dimension_semantics=("arbitrary",)),
    )(x0, x1, x2, W_self0, W_neigh0, W_self1, W_neigh1)


# speedup vs baseline: 12.0125x; 1.0589x over previous
import jax
import jax.numpy as jnp
from jax.experimental import pallas as pl
from jax.experimental.pallas import tpu as pltpu

N = 10000
D = 128
K1 = 5
K2 = 10

B = 400
GRID = N // B


def _fused_body(x0_ref, x1_ref, x2_ref, ws0_ref, wn0_ref, ws1_ref, wn1_ref,
                out_ref):
    f32 = jnp.float32
    ws0 = ws0_ref[...]
    wn0 = wn0_ref[...]

    m1 = None
    mh1 = None
    for j in range(K1):
        x1j = x1_ref[pl.Slice(j, B, K1), :]
        m2j = x2_ref[pl.Slice(K2 * j, B, K1 * K2), :]
        for u in range(1, K2):
            m2j = m2j + x2_ref[pl.Slice(K2 * j + u, B, K1 * K2), :]
        h1j = jnp.maximum(
            jnp.dot(x1j, ws0, preferred_element_type=f32)
            + jnp.dot(m2j * (1.0 / K2), wn0, preferred_element_type=f32),
            0.0)
        m1 = x1j if m1 is None else m1 + x1j
        mh1 = h1j if mh1 is None else mh1 + h1j

    h0 = jnp.maximum(
        jnp.dot(x0_ref[...], ws0, preferred_element_type=f32)
        + jnp.dot(m1 * (1.0 / K1), wn0, preferred_element_type=f32), 0.0)

    out_ref[...] = (
        jnp.dot(h0, ws1_ref[...], preferred_element_type=f32)
        + jnp.dot(mh1 * (1.0 / K1), wn1_ref[...], preferred_element_type=f32))


def kernel(x0, x1, x2, W_self0, W_neigh0, W_self1, W_neigh1):
    w_spec = pl.BlockSpec((D, D), lambda i: (0, 0))
    return pl.pallas_call(
        _fused_body,
        grid=(GRID,),
        in_specs=[
            pl.BlockSpec((B, D), lambda i: (i, 0)),
            pl.BlockSpec((K1 * B, D), lambda i: (i, 0)),
            pl.BlockSpec((K1 * K2 * B, D), lambda i: (i, 0)),
            w_spec, w_spec, w_spec, w_spec,
        ],
        out_specs=pl.BlockSpec((B, D), lambda i: (i, 0)),
        out_shape=jax.ShapeDtypeStruct((N, D), jnp.float32),
        compiler_params=pltpu.CompilerParams(
            dimension_semantics=("arbitrary",)),
    )(x0, x1, x2, W_self0, W_neigh0, W_self1, W_neigh1)
